# baseline (speedup 1.0000x reference)
import functools

import jax
import jax.numpy as jnp
from jax import lax
from jax.experimental import pallas as pl
from jax.experimental.pallas import tpu as pltpu
from jax.experimental.pallas import tpu_sc as plsc

B = 16384
D = 32
BB = 2048


def _sc_gather4(tables, ids):
    info = plsc.get_sparse_core_info()
    nw = info.num_cores * info.num_subcores
    b_per_w = B // nw
    mesh = plsc.VectorSubcoreMesh(core_axis_name="c", subcore_axis_name="s")

    @functools.partial(
        pl.kernel,
        mesh=mesh,
        out_type=jax.ShapeDtypeStruct((4, B, D), jnp.bfloat16),
        scratch_types=[
            pltpu.VMEM((4, b_per_w), jnp.int32),
            pltpu.VMEM((4, b_per_w, D), jnp.bfloat16),
            pltpu.SemaphoreType.DMA,
            pltpu.SemaphoreType.DMA,
            pltpu.SemaphoreType.DMA,
        ],
        compiler_params=pltpu.CompilerParams(use_tc_tiling_on_sc=False),
    )
    def gather_kernel(t0, t1, t2, t3, i0, i1, i2, i3,
                      e_out, idx_v, rows_v, sem_i, sem_g, sem_w):
        wid = lax.axis_index("s") * info.num_cores + lax.axis_index("c")
        base = wid * b_per_w
        tabs = (t0, t1, t2, t3)
        idx_copies = [
            pltpu.async_copy(idx.at[pl.ds(base, b_per_w)], idx_v.at[t],
                             sem_i)
            for t, idx in enumerate((i0, i1, i2, i3))
        ]
        gathers = []
        for t in range(4):
            idx_copies[t].wait()
            gathers.append(
                pltpu.async_copy(tabs[t].at[idx_v.at[t]], rows_v.at[t],
                                 sem_g))
        writes = []
        for t in range(4):
            gathers[t].wait()
            writes.append(
                pltpu.async_copy(
                    rows_v.at[t],
                    e_out.at[t, pl.ds(base, b_per_w), :],
                    sem_w))
        for w in writes:
            w.wait()

    return gather_kernel(*tables, *ids)


def _dot(a, b):
    return jax.lax.dot_general(a, b, (((1,), (0,)), ((), ())),
                               preferred_element_type=jnp.float32)


def _bdot(a, b):
    bf16 = jnp.bfloat16
    return _dot(a.astype(bf16), b.astype(bf16))


def _manifest_body(manifest, W_manifest, b_manifest, out):
    out[...] = (_bdot(manifest[...], W_manifest[...])
                + b_manifest[...]).astype(jnp.bfloat16)


def _manifest_proj(manifest, W_manifest, b_manifest, interpret=False):
    grid = (B // BB,)
    return pl.pallas_call(
        _manifest_body,
        grid=grid,
        in_specs=[
            pl.BlockSpec((BB, 512), lambda i: (i, 0)),
            pl.BlockSpec((512, D), lambda i: (0, 0)),
            pl.BlockSpec((1, D), lambda i: (0, 0)),
        ],
        out_specs=pl.BlockSpec((BB, D), lambda i: (i, 0)),
        out_shape=jax.ShapeDtypeStruct((B, D), jnp.bfloat16),
        compiler_params=pltpu.CompilerParams(
            dimension_semantics=("arbitrary",)),
        interpret=interpret,
    )(manifest, W_manifest, b_manifest.reshape(1, D))


def _mlp_body(scalT, scal, m_emb, emb,
              W_int, b_int, W1, b1, W2, b2, W3, b3, out):
    eps = jnp.float32(1e-8)
    w1 = W1[...]

    def slot(k):
        return w1[k * D:(k + 1) * D, :]

    wi = W_int[...]
    bi = b_int[...]

    acc = _bdot(emb[0], slot(0))
    acc = acc + _bdot(emb[1], slot(3))
    acc = acc + _bdot(emb[2], slot(5))
    acc = acc + _bdot(emb[3], slot(8))
    acc = acc + _bdot(m_emb[...], slot(4))
    for j, k in enumerate((1, 2, 6, 7)):
        full = scalT[j:j + 1, :]
        mn = jnp.min(full)
        mx = jnp.max(full)
        n = (scal[:, j:j + 1] - mn) / (mx - mn + eps)
        s = slot(k)
        acc = acc + n * _dot(wi, s) + _dot(bi, s)
    acc = acc + b1[...]

    h1 = jnp.maximum(acc, 0.0)
    h2 = jnp.maximum(_bdot(h1, W2[...]) + b2[...], 0.0)
    out[...] = _bdot(h2, W3[...]) + b3[...]


def _mlp(scalT, scal, m_emb, emb,
         W_int, b_int, W1, b1, W2, b2, W3, b3, interpret=False):
    grid = (B // BB,)
    full = lambda shape: pl.BlockSpec(shape, lambda i: tuple([0] * len(shape)))
    in_specs = [
        full((4, B)),
        pl.BlockSpec((BB, 4), lambda i: (i, 0)),
        pl.BlockSpec((BB, D), lambda i: (i, 0)),
        pl.BlockSpec((4, BB, D), lambda i: (0, i, 0)),
        full((1, D)), full((1, D)),
        full((9 * D, 256)), full((1, 256)),
        full((256, 64)), full((1, 64)),
        full((64, 1)), full((1, 1)),
    ]
    return pl.pallas_call(
        _mlp_body,
        grid=grid,
        in_specs=in_specs,
        out_specs=pl.BlockSpec((BB, 1), lambda i: (i, 0)),
        out_shape=jax.ShapeDtypeStruct((B, 1), jnp.float32),
        compiler_params=pltpu.CompilerParams(
            dimension_semantics=("arbitrary",)),
        interpret=interpret,
    )(scalT, scal, m_emb, emb,
      W_int, b_int.reshape(1, D),
      W1, b1.reshape(1, 256), W2, b2.reshape(1, 64),
      W3, b3.reshape(1, 1))


def kernel(pod_id, pod_cpu, pod_mem, pod_location, pod_manifest,
           template_resource_id, template_cpu, template_mem,
           template_location, pod_table, template_table, pod_loc_table,
           template_loc_table, W_manifest, b_manifest, W_int, b_int,
           W1, b1, W2, b2, W3, b3):
    i32 = jnp.int32
    bf16 = jnp.bfloat16
    m_emb = _manifest_proj(pod_manifest, W_manifest, b_manifest)
    emb = _sc_gather4(
        (pod_table.astype(bf16), pod_loc_table.astype(bf16),
         template_table.astype(bf16), template_loc_table.astype(bf16)),
        (pod_id.astype(i32), pod_location.astype(i32),
         template_resource_id.astype(i32), template_location.astype(i32)))
    scal = jnp.stack([pod_cpu, pod_mem, template_cpu, template_mem], axis=1)
    scalT = jnp.stack([pod_cpu, pod_mem, template_cpu, template_mem], axis=0)
    return _mlp(scalT, scal, m_emb, emb,
                W_int, b_int, W1, b1, W2, b2, W3, b3)


# speedup vs baseline: 1.2006x; 1.2006x over previous
import functools

import jax
import jax.numpy as jnp
from jax import lax
from jax.experimental import pallas as pl
from jax.experimental.pallas import tpu as pltpu
from jax.experimental.pallas import tpu_sc as plsc

B = 16384
D = 32
BB = 2048


def _sc_gather4(tables, ids):
    info = plsc.get_sparse_core_info()
    nw = info.num_cores * info.num_subcores
    b_per_w = B // nw
    mesh = plsc.VectorSubcoreMesh(core_axis_name="c", subcore_axis_name="s")

    @functools.partial(
        pl.kernel,
        mesh=mesh,
        out_type=jax.ShapeDtypeStruct((B, 4 * D), jnp.bfloat16),
        scratch_types=[
            pltpu.VMEM((4, b_per_w), jnp.int32),
            pltpu.VMEM((4, b_per_w, D), jnp.bfloat16),
            pltpu.SemaphoreType.DMA,
            pltpu.SemaphoreType.DMA,
            pltpu.SemaphoreType.DMA,
        ],
        compiler_params=pltpu.CompilerParams(use_tc_tiling_on_sc=False),
    )
    def gather_kernel(t0, t1, t2, t3, i0, i1, i2, i3,
                      e_out, idx_v, rows_v, sem_i, sem_g, sem_w):
        wid = lax.axis_index("s") * info.num_cores + lax.axis_index("c")
        base = wid * b_per_w
        tabs = (t0, t1, t2, t3)
        idx_copies = [
            pltpu.async_copy(idx.at[pl.ds(base, b_per_w)], idx_v.at[t],
                             sem_i)
            for t, idx in enumerate((i0, i1, i2, i3))
        ]
        gathers = []
        for t in range(4):
            idx_copies[t].wait()
            gathers.append(
                pltpu.async_copy(tabs[t].at[idx_v.at[t]], rows_v.at[t],
                                 sem_g))
        writes = []
        for t in range(4):
            gathers[t].wait()
            writes.append(
                pltpu.async_copy(
                    rows_v.at[t],
                    e_out.at[pl.ds(base, b_per_w), pl.ds(t * D, D)],
                    sem_w))
        for w in writes:
            w.wait()

    return gather_kernel(*tables, *ids)


def _dot(a, b):
    return jax.lax.dot_general(a, b, (((1,), (0,)), ((), ())),
                               preferred_element_type=jnp.float32)


def _bdot(a, b):
    bf16 = jnp.bfloat16
    return _dot(a.astype(bf16), b.astype(bf16))


def _manifest_body(manifest, W_manifest, b_manifest, out):
    out[...] = (_bdot(manifest[...], W_manifest[...])
                + b_manifest[...]).astype(jnp.bfloat16)


def _manifest_proj(manifest, W_manifest_pad, b_manifest_pad,
                   interpret=False):
    grid = (B // BB,)
    return pl.pallas_call(
        _manifest_body,
        grid=grid,
        in_specs=[
            pl.BlockSpec((BB, 512), lambda i: (i, 0)),
            pl.BlockSpec((512, 128), lambda i: (0, 0)),
            pl.BlockSpec((1, 128), lambda i: (0, 0)),
        ],
        out_specs=pl.BlockSpec((BB, 128), lambda i: (i, 0)),
        out_shape=jax.ShapeDtypeStruct((B, 128), jnp.bfloat16),
        compiler_params=pltpu.CompilerParams(
            dimension_semantics=("arbitrary",)),
        interpret=interpret,
    )(manifest, W_manifest_pad, b_manifest_pad)


def _mlp_body(scalT_f, scalT_b, m_emb, emb,
              W_int, b_int, W1, W1sel, W1man, b1, W2, b2, W3, b3, out):
    eps = jnp.float32(1e-8)
    w1 = W1[...]

    def slot(k):
        return w1[k * D:(k + 1) * D, :]

    wi = W_int[...]
    bi = b_int[...]

    acc = _bdot(emb[...], W1sel[...])
    acc = acc + _bdot(m_emb[...], W1man[...])
    s1, s2, s6, s7 = slot(1), slot(2), slot(6), slot(7)
    v = jnp.concatenate(
        [_dot(wi, s1), _dot(wi, s2), _dot(wi, s6), _dot(wi, s7)], axis=0)
    vals = scalT_b[...]
    mn = jnp.min(scalT_f[...], axis=1, keepdims=True)
    mx = jnp.max(scalT_f[...], axis=1, keepdims=True)
    nn = (vals - mn) / (mx - mn + eps)
    acc = acc + jax.lax.dot_general(
        nn, v, (((0,), (0,)), ((), ())),
        preferred_element_type=jnp.float32)
    acc = acc + _dot(bi, s1 + s2 + s6 + s7)
    acc = acc + b1[...]

    h1 = jnp.maximum(acc, 0.0)
    h2 = jnp.maximum(_bdot(h1, W2[...]) + b2[...], 0.0)
    out[...] = _bdot(h2, W3[...]) + b3[...]


def _mlp(scalT, m_emb, emb,
         W_int, b_int, W1, W1sel, W1man, b1, W2, b2, W3, b3,
         interpret=False):
    grid = (B // BB,)
    full = lambda shape: pl.BlockSpec(shape, lambda i: tuple([0] * len(shape)))
    in_specs = [
        full((4, B)),
        pl.BlockSpec((4, BB), lambda i: (0, i)),
        pl.BlockSpec((BB, 128), lambda i: (i, 0)),
        pl.BlockSpec((BB, 128), lambda i: (i, 0)),
        full((1, D)), full((1, D)),
        full((9 * D, 256)), full((4 * D, 256)), full((4 * D, 256)),
        full((1, 256)),
        full((256, 64)), full((1, 64)),
        full((64, 1)), full((1, 1)),
    ]
    return pl.pallas_call(
        _mlp_body,
        grid=grid,
        in_specs=in_specs,
        out_specs=pl.BlockSpec((BB, 1), lambda i: (i, 0)),
        out_shape=jax.ShapeDtypeStruct((B, 1), jnp.float32),
        compiler_params=pltpu.CompilerParams(
            dimension_semantics=("arbitrary",)),
        interpret=interpret,
    )(scalT, scalT, m_emb, emb,
      W_int, b_int.reshape(1, D),
      W1, W1sel, W1man, b1.reshape(1, 256), W2, b2.reshape(1, 64),
      W3, b3.reshape(1, 1))


def kernel(pod_id, pod_cpu, pod_mem, pod_location, pod_manifest,
           template_resource_id, template_cpu, template_mem,
           template_location, pod_table, template_table, pod_loc_table,
           template_loc_table, W_manifest, b_manifest, W_int, b_int,
           W1, b1, W2, b2, W3, b3):
    i32 = jnp.int32
    bf16 = jnp.bfloat16
    f32 = jnp.float32
    Wm_pad = jnp.pad(W_manifest, ((0, 0), (0, 128 - D)))
    bm_pad = jnp.pad(b_manifest, (0, 128 - D)).reshape(1, 128)
    W1sel = jnp.concatenate(
        [W1[0 * D:1 * D], W1[3 * D:4 * D], W1[5 * D:6 * D], W1[8 * D:9 * D]],
        axis=0)
    W1man = jnp.pad(W1[4 * D:5 * D], ((0, 128 - D), (0, 0)))

    m_emb = _manifest_proj(pod_manifest, Wm_pad, bm_pad)
    emb = _sc_gather4(
        (pod_table.astype(bf16), pod_loc_table.astype(bf16),
         template_table.astype(bf16), template_loc_table.astype(bf16)),
        (pod_id.astype(i32), pod_location.astype(i32),
         template_resource_id.astype(i32), template_location.astype(i32)))
    scalT = jnp.stack(
        [pod_cpu, pod_mem, template_cpu, template_mem], axis=0)
    return _mlp(scalT, m_emb, emb,
                W_int, b_int, W1, W1sel, W1man, b1, W2, b2, W3, b3)


# speedup vs baseline: 1.4030x; 1.1686x over previous
import functools

import jax
import jax.numpy as jnp
from jax import lax
from jax.experimental import pallas as pl
from jax.experimental.pallas import tpu as pltpu
from jax.experimental.pallas import tpu_sc as plsc

B = 16384
D = 32
BB = 2048


def _sc_gather4(tables, ids):
    info = plsc.get_sparse_core_info()
    nw = info.num_cores * info.num_subcores
    b_per_w = B // nw
    mesh = plsc.VectorSubcoreMesh(core_axis_name="c", subcore_axis_name="s")

    @functools.partial(
        pl.kernel,
        mesh=mesh,
        out_type=jax.ShapeDtypeStruct((B, 4 * D), jnp.float32),
        scratch_types=[
            pltpu.VMEM((4, b_per_w), jnp.int32),
            pltpu.VMEM((4, b_per_w, D), jnp.float32),
            pltpu.SemaphoreType.DMA,
            pltpu.SemaphoreType.DMA,
            pltpu.SemaphoreType.DMA,
        ],
        compiler_params=pltpu.CompilerParams(use_tc_tiling_on_sc=False),
    )
    def gather_kernel(t0, t1, t2, t3, i0, i1, i2, i3,
                      e_out, idx_v, rows_v, sem_i, sem_g, sem_w):
        wid = lax.axis_index("s") * info.num_cores + lax.axis_index("c")
        base = wid * b_per_w
        tabs = (t0, t1, t2, t3)
        idx_copies = [
            pltpu.async_copy(idx.at[pl.ds(base, b_per_w)], idx_v.at[t],
                             sem_i)
            for t, idx in enumerate((i0, i1, i2, i3))
        ]
        gathers = []
        for t in range(4):
            idx_copies[t].wait()
            gathers.append(
                pltpu.async_copy(tabs[t].at[idx_v.at[t]], rows_v.at[t],
                                 sem_g))
        writes = []
        for t in range(4):
            gathers[t].wait()
            writes.append(
                pltpu.async_copy(
                    rows_v.at[t],
                    e_out.at[pl.ds(base, b_per_w), pl.ds(t * D, D)],
                    sem_w))
        for w in writes:
            w.wait()

    return gather_kernel(*tables, *ids)


def _dot(a, b):
    return jax.lax.dot_general(a, b, (((1,), (0,)), ((), ())),
                               preferred_element_type=jnp.float32)


def _bdot(a, b):
    bf16 = jnp.bfloat16
    return _dot(a.astype(bf16), b.astype(bf16))


def _manifest_body(manifest, W_manifest, b_manifest, out):
    out[...] = (_bdot(manifest[...], W_manifest[...])
                + b_manifest[...]).astype(jnp.bfloat16)


def _manifest_proj(manifest, W_manifest_pad, b_manifest_pad,
                   interpret=False):
    grid = (B // BB,)
    return pl.pallas_call(
        _manifest_body,
        grid=grid,
        in_specs=[
            pl.BlockSpec((BB, 512), lambda i: (i, 0)),
            pl.BlockSpec((512, 128), lambda i: (0, 0)),
            pl.BlockSpec((1, 128), lambda i: (0, 0)),
        ],
        out_specs=pl.BlockSpec((BB, 128), lambda i: (i, 0)),
        out_shape=jax.ShapeDtypeStruct((B, 128), jnp.bfloat16),
        compiler_params=pltpu.CompilerParams(
            dimension_semantics=("arbitrary",)),
        interpret=interpret,
    )(manifest, W_manifest_pad, b_manifest_pad)


def _mlp_body(scalT_f, scalT_b, m_emb, emb,
              W_int, b_int, W1, W1sel, W1man, b1, W2, b2, W3, b3, out):
    eps = jnp.float32(1e-8)
    w1 = W1[...]

    def slot(k):
        return w1[k * D:(k + 1) * D, :]

    wi = W_int[...]
    bi = b_int[...]

    acc = _bdot(emb[...], W1sel[...])
    acc = acc + _bdot(m_emb[...], W1man[...])
    s1, s2, s6, s7 = slot(1), slot(2), slot(6), slot(7)
    v = jnp.concatenate(
        [_dot(wi, s1), _dot(wi, s2), _dot(wi, s6), _dot(wi, s7)], axis=0)
    vals = scalT_b[...]
    mn = jnp.min(scalT_f[...], axis=1, keepdims=True)
    mx = jnp.max(scalT_f[...], axis=1, keepdims=True)
    nn = (vals - mn) / (mx - mn + eps)
    acc = acc + jax.lax.dot_general(
        nn, v, (((0,), (0,)), ((), ())),
        preferred_element_type=jnp.float32)
    acc = acc + _dot(bi, s1 + s2 + s6 + s7)
    acc = acc + b1[...]

    h1 = jnp.maximum(acc, 0.0)
    h2 = jnp.maximum(_bdot(h1, W2[...]) + b2[...], 0.0)
    out[...] = _bdot(h2, W3[...]) + b3[...]


def _mlp(scalT, m_emb, emb,
         W_int, b_int, W1, W1sel, W1man, b1, W2, b2, W3, b3,
         interpret=False):
    grid = (B // BB,)
    full = lambda shape: pl.BlockSpec(shape, lambda i: tuple([0] * len(shape)))
    in_specs = [
        full((4, B)),
        pl.BlockSpec((4, BB), lambda i: (0, i)),
        pl.BlockSpec((BB, 128), lambda i: (i, 0)),
        pl.BlockSpec((BB, 128), lambda i: (i, 0)),
        full((1, D)), full((1, D)),
        full((9 * D, 256)), full((4 * D, 256)), full((4 * D, 256)),
        full((1, 256)),
        full((256, 64)), full((1, 64)),
        full((64, 1)), full((1, 1)),
    ]
    return pl.pallas_call(
        _mlp_body,
        grid=grid,
        in_specs=in_specs,
        out_specs=pl.BlockSpec((BB, 1), lambda i: (i, 0)),
        out_shape=jax.ShapeDtypeStruct((B, 1), jnp.float32),
        compiler_params=pltpu.CompilerParams(
            dimension_semantics=("arbitrary",)),
        interpret=interpret,
    )(scalT, scalT, m_emb, emb,
      W_int, b_int.reshape(1, D),
      W1, W1sel, W1man, b1.reshape(1, 256), W2, b2.reshape(1, 64),
      W3, b3.reshape(1, 1))


def kernel(pod_id, pod_cpu, pod_mem, pod_location, pod_manifest,
           template_resource_id, template_cpu, template_mem,
           template_location, pod_table, template_table, pod_loc_table,
           template_loc_table, W_manifest, b_manifest, W_int, b_int,
           W1, b1, W2, b2, W3, b3):
    i32 = jnp.int32
    bf16 = jnp.bfloat16
    f32 = jnp.float32
    Wm_pad = jnp.pad(W_manifest, ((0, 0), (0, 128 - D)))
    bm_pad = jnp.pad(b_manifest, (0, 128 - D)).reshape(1, 128)
    W1sel = jnp.concatenate(
        [W1[0 * D:1 * D], W1[3 * D:4 * D], W1[5 * D:6 * D], W1[8 * D:9 * D]],
        axis=0)
    W1man = jnp.pad(W1[4 * D:5 * D], ((0, 128 - D), (0, 0)))

    m_emb = _manifest_proj(pod_manifest, Wm_pad, bm_pad)
    emb = _sc_gather4(
        (pod_table, pod_loc_table, template_table, template_loc_table),
        (pod_id.astype(i32), pod_location.astype(i32),
         template_resource_id.astype(i32), template_location.astype(i32)))
    scalT = jnp.stack(
        [pod_cpu, pod_mem, template_cpu, template_mem], axis=0)
    return _mlp(scalT, m_emb, emb,
                W_int, b_int, W1, W1sel, W1man, b1, W2, b2, W3, b3)
